# baseline (speedup 1.0000x reference)
import jax
import jax.numpy as jnp
from jax.experimental import pallas as pl
from jax.experimental.pallas import tpu as pltpu

BATCH = 16
DIM = 768
NH = 12
HD = 64
NLOC = 1024
NREG = 64
RS = 16
TOPK = 32
OUT_DIM = 2 * DIM
SCALE = DIM ** -0.5


_NT = (((1,), (1,)), ((), ()))


def _rot(t, s):
    return pltpu.roll(t, NLOC - s, 1)


def _bra_kernel(xs_ref, x0_ref, wq_ref, bq_ref, wkv_ref, bv_ref,
                wo_ref, bo_ref, out_ref):
    f32 = jnp.float32
    xs = xs_ref[0]
    x0 = x0_ref[0]
    q = jax.lax.dot_general(x0, wq_ref[...], _NT,
                            preferred_element_type=f32) + bq_ref[...]

    k = jnp.dot(wkv_ref[0:DIM, :], xs, preferred_element_type=f32)

    m = jnp.maximum(k, _rot(k, 1))
    m = jnp.maximum(m, _rot(m, 2))
    m = jnp.maximum(m, _rot(m, 32))
    m = jnp.maximum(m, _rot(m, 64))

    le = jax.lax.broadcasted_iota(jnp.int32, (NLOC, NREG), 0)
    re = jax.lax.broadcasted_iota(jnp.int32, (NLOC, NREG), 1)
    emat = (((le % 4) == 0) & (((le // 32) % 4) == 0)
            & (((le // 128) * 8 + (le % 32) // 4) == re)).astype(f32)
    kr = jnp.dot(m, emat, preferred_element_type=f32)
    a = jnp.dot(q, kr, preferred_element_type=f32)

    arow = jnp.broadcast_to(a, (NREG, NREG))
    acol = arow.T
    ii = jax.lax.broadcasted_iota(jnp.int32, (NREG, NREG), 0)
    jj = jax.lax.broadcasted_iota(jnp.int32, (NREG, NREG), 1)
    beats = (acol > arow) | ((acol == arow) & (ii < jj))
    rank = jnp.sum(beats.astype(f32), axis=0, keepdims=True)
    l_i = jax.lax.broadcasted_iota(jnp.int32, (NREG, NLOC), 1)
    r_i = jax.lax.broadcasted_iota(jnp.int32, (NREG, NLOC), 0)
    rmap = (((l_i // 128) * 8 + (l_i % 32) // 4) == r_i).astype(f32)
    rankloc = jnp.dot(rank, rmap, preferred_element_type=f32)
    maskloc = rankloc < float(TOPK)

    hidx = jax.lax.broadcasted_iota(jnp.int32, (NH, DIM), 0)
    cidx = jax.lax.broadcasted_iota(jnp.int32, (NH, DIM), 1)
    diag = (cidx // HD) == hidx
    qm = jnp.where(diag, jnp.broadcast_to(q, (NH, DIM)), 0.0)
    s = jnp.dot(qm, k, preferred_element_type=f32) * SCALE
    s = jnp.where(maskloc, s, -1e30)
    s = s - jnp.max(s, axis=1, keepdims=True)
    e = jnp.exp(s)
    attn = e / jnp.sum(e, axis=1, keepdims=True)

    z = jax.lax.dot_general(attn, xs, (((1,), (1,)), ((), ())),
                            preferred_element_type=f32)

    o_full = jax.lax.dot_general(z, wkv_ref[DIM:, :], _NT,
                                 preferred_element_type=f32)
    o_vec = jnp.sum(jnp.where(diag, o_full, 0.0), axis=0,
                    keepdims=True) + bv_ref[...]

    out_ref[0] = (jax.lax.dot_general(o_vec, wo_ref[...], _NT,
                                      preferred_element_type=f32)
                  + bo_ref[...])


def kernel(x, Wq, bq, Wkv, bkv, Wo, bo):
    xs = x[:, 1:].reshape(BATCH, DIM, NLOC)
    x0 = x[:, 0].reshape(BATCH, 1, DIM)
    bq2 = bq.reshape(1, DIM)
    bv2 = bkv[DIM:].reshape(1, DIM)
    bo2 = bo.reshape(1, OUT_DIM)

    out = pl.pallas_call(
        _bra_kernel,
        grid=(BATCH,),
        in_specs=[
            pl.BlockSpec((1, DIM, NLOC), lambda b: (b, 0, 0)),
            pl.BlockSpec((1, 1, DIM), lambda b: (b, 0, 0)),
            pl.BlockSpec((DIM, DIM), lambda b: (0, 0)),
            pl.BlockSpec((1, DIM), lambda b: (0, 0)),
            pl.BlockSpec((OUT_DIM, DIM), lambda b: (0, 0)),
            pl.BlockSpec((1, DIM), lambda b: (0, 0)),
            pl.BlockSpec((OUT_DIM, DIM), lambda b: (0, 0)),
            pl.BlockSpec((1, OUT_DIM), lambda b: (0, 0)),
        ],
        out_specs=pl.BlockSpec((1, 1, OUT_DIM), lambda b: (b, 0, 0)),
        out_shape=jax.ShapeDtypeStruct((BATCH, 1, OUT_DIM), jnp.float32),
    )(xs, x0, Wq, bq2, Wkv, bv2, Wo, bo2)
    return out.reshape(BATCH, OUT_DIM)


# speedup vs baseline: 1.5620x; 1.5620x over previous
import jax
import jax.numpy as jnp
from jax.experimental import pallas as pl
from jax.experimental.pallas import tpu as pltpu

BATCH = 16
DIM = 768
NH = 12
HD = 64
NLOC = 1024
NREG = 64
RS = 16
TOPK = 32
OUT_DIM = 2 * DIM
SCALE = DIM ** -0.5

_NT = (((1,), (1,)), ((), ()))


def _rot(t, s):
    return pltpu.roll(t, NLOC - s, 1)


def _bra_kernel(xs_ref, x0_ref, wq_ref, bq_ref, wkv_ref, bv_ref,
                wo_ref, bo_ref, out_ref):
    f32 = jnp.float32
    bf16 = jnp.bfloat16
    xs = xs_ref[0]
    x0 = x0_ref[0]
    q = jax.lax.dot_general(x0, wq_ref[...], _NT,
                            preferred_element_type=f32) + bq_ref[...]

    k = jnp.dot(wkv_ref[0:DIM, :], xs,
                preferred_element_type=f32)
    kb = k.astype(bf16)

    m = jnp.maximum(kb, _rot(kb, 1))
    m = jnp.maximum(m, _rot(m, 2))
    m = jnp.maximum(m, _rot(m, 32))
    m = jnp.maximum(m, _rot(m, 64))

    le = jax.lax.broadcasted_iota(jnp.int32, (NLOC, NREG), 0)
    re = jax.lax.broadcasted_iota(jnp.int32, (NLOC, NREG), 1)
    emat = (((le % 4) == 0) & (((le // 32) % 4) == 0)
            & (((le // 128) * 8 + (le % 32) // 4) == re)).astype(bf16)
    kr = jnp.dot(m, emat, preferred_element_type=f32)
    a = jnp.dot(q.astype(bf16), kr.astype(bf16),
                preferred_element_type=f32)

    arow = jnp.broadcast_to(a, (NREG, NREG))
    acol = arow.T
    ii = jax.lax.broadcasted_iota(jnp.int32, (NREG, NREG), 0)
    jj = jax.lax.broadcasted_iota(jnp.int32, (NREG, NREG), 1)
    beats = (acol > arow) | ((acol == arow) & (ii < jj))
    rank = jnp.sum(beats.astype(f32), axis=0, keepdims=True)
    l_i = jax.lax.broadcasted_iota(jnp.int32, (NREG, NLOC), 1)
    r_i = jax.lax.broadcasted_iota(jnp.int32, (NREG, NLOC), 0)
    rmap = (((l_i // 128) * 8 + (l_i % 32) // 4) == r_i).astype(bf16)
    rankloc = jnp.dot(rank.astype(bf16), rmap,
                      preferred_element_type=f32)
    maskloc = rankloc < float(TOPK)

    hidx = jax.lax.broadcasted_iota(jnp.int32, (NH, DIM), 0)
    cidx = jax.lax.broadcasted_iota(jnp.int32, (NH, DIM), 1)
    diag = (cidx // HD) == hidx
    qm = jnp.where(diag, jnp.broadcast_to(q, (NH, DIM)), 0.0).astype(bf16)
    s = jnp.dot(qm, kb, preferred_element_type=f32) * SCALE
    s = jnp.where(maskloc, s, -1e30)
    s = s - jnp.max(s, axis=1, keepdims=True)
    e = jnp.exp(s)
    attn = e / jnp.sum(e, axis=1, keepdims=True)

    z = jax.lax.dot_general(attn.astype(bf16), xs, _NT,
                            preferred_element_type=f32)

    o_full = jax.lax.dot_general(z.astype(bf16), wkv_ref[DIM:, :], _NT,
                                 preferred_element_type=f32)
    o_vec = jnp.sum(jnp.where(diag, o_full, 0.0), axis=0,
                    keepdims=True) + bv_ref[...]

    out_ref[0] = (jax.lax.dot_general(o_vec.astype(bf16), wo_ref[...], _NT,
                                      preferred_element_type=f32)
                  + bo_ref[...])


def kernel(x, Wq, bq, Wkv, bkv, Wo, bo):
    bf16 = jnp.bfloat16
    xs = x[:, 1:].reshape(BATCH, DIM, NLOC).astype(bf16)
    x0 = x[:, 0].reshape(BATCH, 1, DIM).astype(bf16)
    bq2 = bq.reshape(1, DIM)
    bv2 = bkv[DIM:].reshape(1, DIM)
    bo2 = bo.reshape(1, OUT_DIM)

    out = pl.pallas_call(
        _bra_kernel,
        grid=(BATCH,),
        in_specs=[
            pl.BlockSpec((1, DIM, NLOC), lambda b: (b, 0, 0)),
            pl.BlockSpec((1, 1, DIM), lambda b: (b, 0, 0)),
            pl.BlockSpec((DIM, DIM), lambda b: (0, 0)),
            pl.BlockSpec((1, DIM), lambda b: (0, 0)),
            pl.BlockSpec((OUT_DIM, DIM), lambda b: (0, 0)),
            pl.BlockSpec((1, DIM), lambda b: (0, 0)),
            pl.BlockSpec((OUT_DIM, DIM), lambda b: (0, 0)),
            pl.BlockSpec((1, OUT_DIM), lambda b: (0, 0)),
        ],
        out_specs=pl.BlockSpec((1, 1, OUT_DIM), lambda b: (b, 0, 0)),
        out_shape=jax.ShapeDtypeStruct((BATCH, 1, OUT_DIM), jnp.float32),
        compiler_params=pltpu.CompilerParams(
            dimension_semantics=("parallel",)),
    )(xs, x0, Wq.astype(bf16), bq2, Wkv.astype(bf16), bv2,
      Wo.astype(bf16), bo2)
    return out.reshape(BATCH, OUT_DIM)
